# baseline (speedup 1.0000x reference)
import functools

import jax
import jax.numpy as jnp
from jax import lax
from jax.experimental import pallas as pl
from jax.experimental.pallas import tpu as pltpu
from jax.experimental.pallas import tpu_sc as plsc

EMBED_DIM = 32
BATCH = 16384
CHUNK = 128


def _make_sc_kernel():
    info = plsc.get_sparse_core_info()
    nc, ns = info.num_cores, info.num_subcores
    nw = nc * ns
    b_per_w = BATCH // nw
    n_chunks = b_per_w // CHUNK
    groups = b_per_w // 16

    mesh = plsc.VectorSubcoreMesh(core_axis_name="c", subcore_axis_name="s")

    @functools.partial(
        pl.kernel,
        mesh=mesh,
        out_type=jax.ShapeDtypeStruct((BATCH,), jnp.float32),
        compiler_params=pltpu.CompilerParams(
            needs_layout_passes=False, use_tc_tiling_on_sc=False),
        scratch_types=[
            pltpu.VMEM((n_chunks, CHUNK), jnp.int32),
            pltpu.VMEM((n_chunks, CHUNK), jnp.int32),
            pltpu.VMEM((b_per_w, EMBED_DIM), jnp.float32),
            pltpu.VMEM((b_per_w, EMBED_DIM), jnp.float32),
            pltpu.VMEM((b_per_w,), jnp.float32),
            pltpu.VMEM((b_per_w,), jnp.float32),
            pltpu.VMEM((16,), jnp.float32),
            pltpu.VMEM((b_per_w,), jnp.float32),
            pltpu.SemaphoreType.DMA,
        ],
    )
    def k(uidx_hbm, iidx_hbm, uemb_hbm, iemb_hbm, ubias_hbm, ibias_hbm,
          gbias_hbm, out_hbm, uidx_v, iidx_v, urows_v, irows_v,
          ub_v, ib_v, gb_v, out_v, sem):
        wid = lax.axis_index("s") * nc + lax.axis_index("c")
        base = wid * b_per_w
        row0 = wid * n_chunks

        pltpu.sync_copy(uidx_hbm.at[pl.ds(row0, n_chunks)], uidx_v)
        pltpu.sync_copy(iidx_hbm.at[pl.ds(row0, n_chunks)], iidx_v)
        pltpu.sync_copy(gbias_hbm, gb_v)

        copies = []
        for j in range(n_chunks):
            dst = pl.ds(j * CHUNK, CHUNK)
            copies.append(pltpu.async_copy(
                uemb_hbm.at[uidx_v.at[j]], urows_v.at[dst], sem))
            copies.append(pltpu.async_copy(
                iemb_hbm.at[iidx_v.at[j]], irows_v.at[dst], sem))
            copies.append(pltpu.async_copy(
                ubias_hbm.at[uidx_v.at[j]], ub_v.at[dst], sem))
            copies.append(pltpu.async_copy(
                ibias_hbm.at[iidx_v.at[j]], ib_v.at[dst], sem))
        for c in copies:
            c.wait()

        lane = lax.iota(jnp.int32, 16)
        gb = gb_v[...]

        def body(g, carry):
            rowbase = g * 16
            rows = rowbase + lane
            acc = gb + ub_v[pl.ds(rowbase, 16)] + ib_v[pl.ds(rowbase, 16)]
            for d in range(EMBED_DIM):
                cols = jnp.full((16,), d, jnp.int32)
                u = plsc.load_gather(urows_v, [rows, cols])
                q = plsc.load_gather(irows_v, [rows, cols])
                acc = acc + u * q
            out_v[pl.ds(rowbase, 16)] = acc
            return carry

        lax.fori_loop(0, groups, body, 0)
        pltpu.sync_copy(out_v, out_hbm.at[pl.ds(base, b_per_w)])

    return k


def kernel(user_idx, item_idx, user_emb, item_emb, user_bias, item_bias,
           global_bias):
    num_users = user_emb.shape[0]
    num_items = item_emb.shape[0]
    k = _make_sc_kernel()
    uidx = user_idx.astype(jnp.int32).reshape(BATCH // CHUNK, CHUNK)
    iidx = item_idx.astype(jnp.int32).reshape(BATCH // CHUNK, CHUNK)
    ubias = user_bias.reshape(num_users)
    ibias = item_bias.reshape(num_items)
    gbias = jnp.broadcast_to(global_bias.astype(jnp.float32), (16,))
    return k(uidx, iidx, user_emb, item_emb, ubias, ibias, gbias)


# speedup vs baseline: 1.0009x; 1.0009x over previous
import functools

import jax
import jax.numpy as jnp
from jax import lax
from jax.experimental import pallas as pl
from jax.experimental.pallas import tpu as pltpu
from jax.experimental.pallas import tpu_sc as plsc

EMBED_DIM = 32
BATCH = 16384
CHUNK = 128


def _make_sc_kernel():
    info = plsc.get_sparse_core_info()
    nc, ns = info.num_cores, info.num_subcores
    nw = nc * ns
    b_per_w = BATCH // nw
    n_chunks = b_per_w // CHUNK
    groups = b_per_w // 16

    mesh = plsc.VectorSubcoreMesh(core_axis_name="c", subcore_axis_name="s")

    @functools.partial(
        pl.kernel,
        mesh=mesh,
        out_type=jax.ShapeDtypeStruct((BATCH,), jnp.float32),
        compiler_params=pltpu.CompilerParams(
            needs_layout_passes=False, use_tc_tiling_on_sc=False),
        scratch_types=[
            pltpu.VMEM((b_per_w,), jnp.int32),
            pltpu.VMEM((b_per_w,), jnp.int32),
            pltpu.VMEM((b_per_w, EMBED_DIM), jnp.float32),
            pltpu.VMEM((b_per_w, EMBED_DIM), jnp.float32),
            pltpu.VMEM((b_per_w,), jnp.float32),
            pltpu.VMEM((b_per_w,), jnp.float32),
            pltpu.VMEM((16,), jnp.float32),
            pltpu.VMEM((b_per_w,), jnp.float32),
            pltpu.SemaphoreType.DMA,
        ],
    )
    def k(uidx_hbm, iidx_hbm, uemb_hbm, iemb_hbm, ubias_hbm, ibias_hbm,
          gbias_hbm, out_hbm, uidx_v, iidx_v, urows_v, irows_v,
          ub_v, ib_v, gb_v, out_v, sem):
        wid = lax.axis_index("s") * nc + lax.axis_index("c")
        base = wid * b_per_w

        pltpu.sync_copy(uidx_hbm.at[pl.ds(base, b_per_w)], uidx_v)
        pltpu.sync_copy(iidx_hbm.at[pl.ds(base, b_per_w)], iidx_v)
        pltpu.sync_copy(gbias_hbm, gb_v)

        copies = []
        for j in range(n_chunks):
            sl = pl.ds(j * CHUNK, CHUNK)
            copies.append(pltpu.async_copy(
                uemb_hbm.at[uidx_v.at[sl]], urows_v.at[sl], sem))
            copies.append(pltpu.async_copy(
                iemb_hbm.at[iidx_v.at[sl]], irows_v.at[sl], sem))
            copies.append(pltpu.async_copy(
                ubias_hbm.at[uidx_v.at[sl]], ub_v.at[sl], sem))
            copies.append(pltpu.async_copy(
                ibias_hbm.at[iidx_v.at[sl]], ib_v.at[sl], sem))
        for c in copies:
            c.wait()

        lane = lax.iota(jnp.int32, 16)
        gb = gb_v[...]

        def body(g, carry):
            rowbase = g * 16
            rows = rowbase + lane
            acc = gb + ub_v[pl.ds(rowbase, 16)] + ib_v[pl.ds(rowbase, 16)]
            for d in range(EMBED_DIM):
                cols = jnp.full((16,), d, jnp.int32)
                u = plsc.load_gather(urows_v, [rows, cols])
                q = plsc.load_gather(irows_v, [rows, cols])
                acc = acc + u * q
            out_v[pl.ds(rowbase, 16)] = acc
            return carry

        lax.fori_loop(0, groups, body, 0)
        pltpu.sync_copy(out_v, out_hbm.at[pl.ds(base, b_per_w)])

    return k


def kernel(user_idx, item_idx, user_emb, item_emb, user_bias, item_bias,
           global_bias):
    num_users = user_emb.shape[0]
    num_items = item_emb.shape[0]
    k = _make_sc_kernel()
    uidx = user_idx.astype(jnp.int32)
    iidx = item_idx.astype(jnp.int32)
    ubias = user_bias.reshape(num_users)
    ibias = item_bias.reshape(num_items)
    gbias = jnp.broadcast_to(global_bias.astype(jnp.float32), (16,))
    return k(uidx, iidx, user_emb, item_emb, ubias, ibias, gbias)
